# baseline (speedup 1.0000x reference)
import functools

import jax
import jax.numpy as jnp
from jax import lax
from jax.experimental import pallas as pl
from jax.experimental.pallas import tpu as pltpu
from jax.experimental.pallas import tpu_sc as plsc

NC = 2
NS = 16
NW = NC * NS
L = 16

B = 16384
V = 1000000
R = 8
D = 64
MARGIN = 1.0
K = 2

S = B // NW
CHUNK = 64
NCHUNK = S // CHUNK
GROUPS = CHUNK // L
TP = L + 1
FMT = plsc.PackFormat.INTERLEAVED


def _body(hidx, tidx, jhidx, jtidx, sp, pw, nw,
          ent, relf,
          out_hbm,
          hidx_v, tidx_v, jhidx_v, jtidx_v,
          sp_v, pw_v, nw_v, relf_v,
          ph0, pt0, nh0, nt0, ph1, pt1, nh1, nt1,
          tsp, tsn, out_v, sem0, sem1):
    cid = lax.axis_index("c")
    sid = lax.axis_index("s")
    wid = sid * NC + cid
    base = wid * S

    stage = [
        pltpu.async_copy(hidx.at[pl.ds(base, S)], hidx_v, sem0),
        pltpu.async_copy(tidx.at[pl.ds(base, S)], tidx_v, sem0),
        pltpu.async_copy(jhidx.at[pl.ds(base, S)], jhidx_v, sem0),
        pltpu.async_copy(jtidx.at[pl.ds(base, S)], jtidx_v, sem0),
        pltpu.async_copy(sp.at[pl.ds(base * K, S * K)],
                         sp_v.at[pl.ds(0, S * K)], sem0),
        pltpu.async_copy(pw.at[pl.ds(base, S)], pw_v, sem0),
        pltpu.async_copy(nw.at[pl.ds(base, S)], nw_v, sem0),
        pltpu.async_copy(relf, relf_v, sem0),
    ]
    for dsc in stage:
        dsc.wait()

    bufs = ((ph0, pt0, nh0, nt0), (ph1, pt1, nh1, nt1))
    sems = (sem0, sem1)
    idxs = (hidx_v, tidx_v, jhidx_v, jtidx_v)

    def fire(c):
        p = c % 2
        return [
            pltpu.async_copy(ent.at[idxs[k].at[pl.ds(c * CHUNK, CHUNK)]],
                             bufs[p][k], sems[p])
            for k in range(4)
        ]

    lanes = lax.iota(jnp.int32, L)
    cols = lanes * TP
    zf = jnp.zeros((L,), jnp.float32)

    pending = fire(0)
    total = zf
    for c in range(NCHUNK):
        nxt = fire(c + 1) if c + 1 < NCHUNK else []
        for dsc in pending:
            dsc.wait()
        phb, ptb, nhb, ntb = bufs[c % 2]

        def group_body(g, tot, phb=phb, ptb=ptb, nhb=nhb, ntb=ntb, c=c):
            goff = c * CHUNK + g * L

            def elem_body(i, _):
                e = g * L + i
                sv = sp_v[pl.ds((goff + i) * K, L)]
                pb = sv[0]
                nb = sv[1]
                ap = zf
                an = zf
                for f in range(D // (2 * L)):
                    hp0, hp1 = plsc.unpack(
                        phb[e, pl.ds(f * 2 * L, 2 * L)], format=FMT)
                    tp0, tp1 = plsc.unpack(
                        ptb[e, pl.ds(f * 2 * L, 2 * L)], format=FMT)
                    rp0, rp1 = plsc.unpack(
                        relf_v[pl.ds(pb + f * 2 * L, 2 * L)], format=FMT)
                    hn0, hn1 = plsc.unpack(
                        nhb[e, pl.ds(f * 2 * L, 2 * L)], format=FMT)
                    tn0, tn1 = plsc.unpack(
                        ntb[e, pl.ds(f * 2 * L, 2 * L)], format=FMT)
                    rn0, rn1 = plsc.unpack(
                        relf_v[pl.ds(nb + f * 2 * L, 2 * L)], format=FMT)
                    ap = ap + jnp.abs(hp0 + rp0 - tp0)
                    ap = ap + jnp.abs(hp1 + rp1 - tp1)
                    an = an + jnp.abs(hn0 + rn0 - tn0)
                    an = an + jnp.abs(hn1 + rn1 - tn1)
                tsp[pl.ds(i * TP, L)] = ap
                tsn[pl.ds(i * TP, L)] = an
                return 0

            lax.fori_loop(0, L, elem_body, 0)

            ps = zf
            ns = zf
            for f in range(L):
                ps = ps + plsc.load_gather(tsp, [cols + f])
                ns = ns + plsc.load_gather(tsn, [cols + f])
            pwv = pw_v[pl.ds(goff, L)]
            nwv = nw_v[pl.ds(goff, L)]
            return tot + jnp.maximum(pwv * ps - nwv * ns + MARGIN, 0.0)

        total = lax.fori_loop(0, GROUPS, group_body, total)
        pending = nxt

    out_v[...] = total
    pltpu.sync_copy(out_v, out_hbm.at[pl.ds(wid * L, L)])


_rhine_sc = functools.partial(
    pl.kernel,
    out_type=jax.ShapeDtypeStruct((NW * L,), jnp.float32),
    mesh=plsc.VectorSubcoreMesh(core_axis_name="c", subcore_axis_name="s"),
    compiler_params=pltpu.CompilerParams(
        needs_layout_passes=False, use_tc_tiling_on_sc=False),
    scratch_types=[
        pltpu.VMEM((S,), jnp.int32),
        pltpu.VMEM((S,), jnp.int32),
        pltpu.VMEM((S,), jnp.int32),
        pltpu.VMEM((S,), jnp.int32),
        pltpu.VMEM((S * K + L,), jnp.int32),
        pltpu.VMEM((S,), jnp.float32),
        pltpu.VMEM((S,), jnp.float32),
        pltpu.VMEM((R * D,), jnp.bfloat16),
        pltpu.VMEM((CHUNK, D), jnp.bfloat16),
        pltpu.VMEM((CHUNK, D), jnp.bfloat16),
        pltpu.VMEM((CHUNK, D), jnp.bfloat16),
        pltpu.VMEM((CHUNK, D), jnp.bfloat16),
        pltpu.VMEM((CHUNK, D), jnp.bfloat16),
        pltpu.VMEM((CHUNK, D), jnp.bfloat16),
        pltpu.VMEM((CHUNK, D), jnp.bfloat16),
        pltpu.VMEM((CHUNK, D), jnp.bfloat16),
        pltpu.VMEM((L * TP,), jnp.float32),
        pltpu.VMEM((L * TP,), jnp.float32),
        pltpu.VMEM((L,), jnp.float32),
        pltpu.SemaphoreType.DMA,
        pltpu.SemaphoreType.DMA,
    ],
)(_body)


@jax.jit
def _run(pos_h, pos_t, pos_r, pos_w, neg_h, neg_t, neg_r, neg_w,
         ent_emb, rel_emb):
    i32 = lambda x: x.astype(jnp.int32)
    sp = jnp.stack([i32(pos_r) * D, i32(neg_r) * D], axis=1).reshape(B * K)
    out = _rhine_sc(
        i32(pos_h), i32(pos_t), i32(neg_h), i32(neg_t),
        sp,
        pos_w.astype(jnp.float32), neg_w.astype(jnp.float32),
        ent_emb.astype(jnp.bfloat16),
        rel_emb.astype(jnp.bfloat16).reshape(R * D),
    )
    return jnp.sum(out)


def kernel(pos_h, pos_t, pos_r, pos_w, neg_h, neg_t, neg_r, neg_w,
           ent_emb, rel_emb):
    return _run(pos_h, pos_t, pos_r, pos_w, neg_h, neg_t, neg_r, neg_w,
                ent_emb, rel_emb)


# speedup vs baseline: 1.4897x; 1.4897x over previous
import functools

import jax
import jax.numpy as jnp
from jax import lax
from jax.experimental import pallas as pl
from jax.experimental.pallas import tpu as pltpu
from jax.experimental.pallas import tpu_sc as plsc

NC = 2
NS = 16
NW = NC * NS
L = 16

B = 16384
V = 1000000
R = 8
D = 64
MARGIN = 1.0
K = 2

S = B // NW
CHUNK = 64
NCHUNK = S // CHUNK
GROUPS = CHUNK // L
TP = L + 1
W = 2 * D


def _body(hidx, tidx, jhidx, jtidx, sp, pw, nw,
          ent, relf,
          out_hbm,
          hidx_v, tidx_v, jhidx_v, jtidx_v,
          sp_v, pw_v, nw_v, relf_v,
          ph0, pt0, nh0, nt0, ph1, pt1, nh1, nt1,
          tsp, tsn, out_v, sem0, sem1):
    cid = lax.axis_index("c")
    sid = lax.axis_index("s")
    wid = sid * NC + cid
    base = wid * S

    stage = [
        pltpu.async_copy(hidx.at[pl.ds(base, S)], hidx_v, sem0),
        pltpu.async_copy(tidx.at[pl.ds(base, S)], tidx_v, sem0),
        pltpu.async_copy(jhidx.at[pl.ds(base, S)], jhidx_v, sem0),
        pltpu.async_copy(jtidx.at[pl.ds(base, S)], jtidx_v, sem0),
        pltpu.async_copy(sp.at[pl.ds(base * K, S * K)],
                         sp_v.at[pl.ds(0, S * K)], sem0),
        pltpu.async_copy(pw.at[pl.ds(base, S)], pw_v, sem0),
        pltpu.async_copy(nw.at[pl.ds(base, S)], nw_v, sem0),
        pltpu.async_copy(relf, relf_v, sem0),
    ]
    for dsc in stage:
        dsc.wait()

    bufs = ((ph0, pt0, nh0, nt0), (ph1, pt1, nh1, nt1))
    sems = (sem0, sem1)
    idxs = (hidx_v, tidx_v, jhidx_v, jtidx_v)

    def fire(c):
        p = c % 2
        return [
            pltpu.async_copy(ent.at[idxs[k].at[pl.ds(c * CHUNK, CHUNK)]],
                             bufs[p][k], sems[p])
            for k in range(4)
        ]

    lanes = lax.iota(jnp.int32, L)
    cols = lanes * TP
    zf = jnp.zeros((L,), jnp.float32)

    pending = fire(0)
    total = zf
    for c in range(NCHUNK):
        nxt = fire(c + 1) if c + 1 < NCHUNK else []
        for dsc in pending:
            dsc.wait()
        phb, ptb, nhb, ntb = bufs[c % 2]

        def group_body(g, tot, phb=phb, ptb=ptb, nhb=nhb, ntb=ntb, c=c):
            goff = c * CHUNK + g * L

            def elem_body(i, _):
                e = g * L + i
                sv = sp_v[pl.ds((goff + i) * K, L)]
                pb = sv[0]
                nb = sv[1]
                ap = zf
                an = zf
                for f in range(D // L):
                    hp = phb[e, pl.ds(f * L, L)]
                    tp = ptb[e, pl.ds(f * L, L)]
                    rp = relf_v[pl.ds(pb + f * L, L)]
                    hn = nhb[e, pl.ds(f * L, L)]
                    tn = ntb[e, pl.ds(f * L, L)]
                    rn = relf_v[pl.ds(nb + f * L, L)]
                    ap = ap + jnp.abs(hp + rp - tp)
                    an = an + jnp.abs(hn + rn - tn)
                tsp[pl.ds(i * TP, L)] = ap
                tsn[pl.ds(i * TP, L)] = an
                return 0

            lax.fori_loop(0, L, elem_body, 0)

            ps = zf
            ns = zf
            for f in range(L):
                ps = ps + plsc.load_gather(tsp, [cols + f])
                ns = ns + plsc.load_gather(tsn, [cols + f])
            pwv = pw_v[pl.ds(goff, L)]
            nwv = nw_v[pl.ds(goff, L)]
            return tot + jnp.maximum(pwv * ps - nwv * ns + MARGIN, 0.0)

        total = lax.fori_loop(0, GROUPS, group_body, total)
        pending = nxt

    out_v[...] = total
    pltpu.sync_copy(out_v, out_hbm.at[pl.ds(wid * L, L)])


_rhine_sc = functools.partial(
    pl.kernel,
    out_type=jax.ShapeDtypeStruct((NW * L,), jnp.float32),
    mesh=plsc.VectorSubcoreMesh(core_axis_name="c", subcore_axis_name="s"),
    compiler_params=pltpu.CompilerParams(
        needs_layout_passes=False, use_tc_tiling_on_sc=True),
    scratch_types=[
        pltpu.VMEM((S,), jnp.int32),
        pltpu.VMEM((S,), jnp.int32),
        pltpu.VMEM((S,), jnp.int32),
        pltpu.VMEM((S,), jnp.int32),
        pltpu.VMEM((S * K + L,), jnp.int32),
        pltpu.VMEM((S,), jnp.float32),
        pltpu.VMEM((S,), jnp.float32),
        pltpu.VMEM((R * D,), jnp.float32),
        pltpu.VMEM((CHUNK, W), jnp.float32),
        pltpu.VMEM((CHUNK, W), jnp.float32),
        pltpu.VMEM((CHUNK, W), jnp.float32),
        pltpu.VMEM((CHUNK, W), jnp.float32),
        pltpu.VMEM((CHUNK, W), jnp.float32),
        pltpu.VMEM((CHUNK, W), jnp.float32),
        pltpu.VMEM((CHUNK, W), jnp.float32),
        pltpu.VMEM((CHUNK, W), jnp.float32),
        pltpu.VMEM((L * TP,), jnp.float32),
        pltpu.VMEM((L * TP,), jnp.float32),
        pltpu.VMEM((L,), jnp.float32),
        pltpu.SemaphoreType.DMA,
        pltpu.SemaphoreType.DMA,
    ],
)(_body)


@jax.jit
def _run(pos_h, pos_t, pos_r, pos_w, neg_h, neg_t, neg_r, neg_w,
         ent_emb, rel_emb):
    i32 = lambda x: x.astype(jnp.int32)
    sp = jnp.stack([i32(pos_r) * D, i32(neg_r) * D], axis=1).reshape(B * K)
    out = _rhine_sc(
        i32(pos_h), i32(pos_t), i32(neg_h), i32(neg_t),
        sp,
        pos_w.astype(jnp.float32), neg_w.astype(jnp.float32),
        jnp.pad(ent_emb, ((0, 0), (0, D))),
        rel_emb.astype(jnp.float32).reshape(R * D),
    )
    return jnp.sum(out)


def kernel(pos_h, pos_t, pos_r, pos_w, neg_h, neg_t, neg_r, neg_w,
           ent_emb, rel_emb):
    return _run(pos_h, pos_t, pos_r, pos_w, neg_h, neg_t, neg_r, neg_w,
                ent_emb, rel_emb)
